# baseline (speedup 1.0000x reference)
import functools

import jax
import jax.numpy as jnp
from jax import lax
from jax.experimental import pallas as pl
from jax.experimental.pallas import tpu as pltpu
from jax.experimental.pallas import tpu_sc as plsc

_LANES = 16
_NW = 32
_NC = 2
_CHUNK = 80
_RING = 4


@functools.partial(jax.jit, static_argnames=("n_edges", "n_chunks"))
def _decode(z, src2d, dst2d, *, n_edges, n_chunks):
    mesh = plsc.VectorSubcoreMesh(core_axis_name="c", subcore_axis_name="s")

    @functools.partial(
        pl.kernel,
        mesh=mesh,
        compiler_params=pltpu.CompilerParams(needs_layout_passes=False,
                                             use_tc_tiling_on_sc=False),
        out_type=jax.ShapeDtypeStruct((_NW, n_chunks, _CHUNK), jnp.float32),
        scratch_types=dict(
            src_idx=pltpu.VMEM((n_chunks, _CHUNK), jnp.int32),
            dst_idx=pltpu.VMEM((n_chunks, _CHUNK), jnp.int32),
            rs_bufs=[pltpu.VMEM((_CHUNK, 64), jnp.int32) for _ in range(_RING)],
            rd_bufs=[pltpu.VMEM((_CHUNK, 64), jnp.int32) for _ in range(_RING)],
            out_v=pltpu.VMEM((n_chunks, _CHUNK), jnp.float32),
            sems=[pltpu.SemaphoreType.DMA for _ in range(_RING)],
        ),
    )
    def body(z_hbm, src_hbm, dst_hbm, out_hbm,
             src_idx, dst_idx, rs_bufs, rd_bufs, out_v, sems):
        wid = lax.axis_index("s") * _NC + lax.axis_index("c")

        pltpu.sync_copy(src_hbm.at[wid], src_idx)
        pltpu.sync_copy(dst_hbm.at[wid], dst_idx)

        def issue(l, b):
            pltpu.async_copy(z_hbm.at[src_idx.at[l]], rs_bufs[b], sems[b])
            pltpu.async_copy(z_hbm.at[dst_idx.at[l]], rd_bufs[b], sems[b])

        def drain(l, b):
            pltpu.make_async_copy(z_hbm.at[src_idx.at[l]], rs_bufs[b], sems[b]).wait()
            pltpu.make_async_copy(z_hbm.at[dst_idx.at[l]], rd_bufs[b], sems[b]).wait()

        def compute(l, rs, rd):
            out_row = out_v.at[l]
            zero = jnp.zeros((_LANES,), jnp.float32)
            iota = lax.iota(jnp.int32, _LANES)
            for g in range(_CHUNK // _LANES):
                row = g * _LANES + iota

                def dstep(i, accs):
                    a0, a1, a2, a3 = accs
                    cb = jnp.full((_LANES,), i * 16, jnp.int32) + iota
                    prods = []
                    for k in range(16):
                        col = (cb + k) & 63
                        s = plsc.load_gather(rs, [row, col])
                        t = plsc.load_gather(rd, [row, col])
                        p = (plsc.bitcast(s, jnp.bfloat16)
                             * plsc.bitcast(t, jnp.bfloat16))
                        p0, p1 = plsc.unpack(p, format=plsc.PackFormat.INTERLEAVED)
                        prods.append(p0 + p1)
                    for k in range(0, 16, 4):
                        a0 = a0 + prods[k]
                        a1 = a1 + prods[k + 1]
                        a2 = a2 + prods[k + 2]
                        a3 = a3 + prods[k + 3]
                    return a0, a1, a2, a3

                a0, a1, a2, a3 = lax.fori_loop(0, 64 // 16, dstep,
                                               (zero, zero, zero, zero))
                acc = (a0 + a1) + (a2 + a3)
                out_row[pl.ds(g * _LANES, _LANES)] = 1.0 / (1.0 + jnp.exp(-acc))

        for b in range(_RING):
            issue(b, b)

        def quad(j, carry):
            for b in range(_RING):
                l = _RING * j + b
                drain(l, b)
                compute(l, rs_bufs[b], rd_bufs[b])

                @pl.when(l + _RING < n_chunks)
                def _():
                    issue(l + _RING, b)

            return carry

        lax.fori_loop(0, n_chunks // _RING, quad, 0)
        last = n_chunks - 1
        drain(last, last % _RING)
        compute(last, rs_bufs[last % _RING], rd_bufs[last % _RING])

        pltpu.sync_copy(out_v, out_hbm.at[wid])

    return body(z, src2d, dst2d)


def kernel(z, edge_index):
    n_edges = edge_index.shape[1]
    n_chunks = n_edges // (_NW * _CHUNK)
    ei = edge_index.astype(jnp.int32)
    src3d = ei[0].reshape(_NW, n_chunks, _CHUNK)
    dst3d = ei[1].reshape(_NW, n_chunks, _CHUNK)
    n_nodes, d_model = z.shape
    z_pk = jax.lax.bitcast_convert_type(
        z.astype(jnp.bfloat16).reshape(n_nodes, d_model // 2, 2), jnp.int32)
    out3d = _decode(z_pk, src3d, dst3d, n_edges=n_edges, n_chunks=n_chunks)
    return out3d.reshape(n_edges)


# speedup vs baseline: 1.2371x; 1.2371x over previous
import functools

import jax
import jax.numpy as jnp
from jax import lax
from jax.experimental import pallas as pl
from jax.experimental.pallas import tpu as pltpu
from jax.experimental.pallas import tpu_sc as plsc

_LANES = 16
_NW = 32
_NC = 2
_CHUNK = 400
_RING = 2


@functools.partial(jax.jit, static_argnames=("n_edges", "n_chunks"))
def _decode(z, src2d, dst2d, *, n_edges, n_chunks):
    mesh = plsc.VectorSubcoreMesh(core_axis_name="c", subcore_axis_name="s")

    @functools.partial(
        pl.kernel,
        mesh=mesh,
        compiler_params=pltpu.CompilerParams(needs_layout_passes=False,
                                             use_tc_tiling_on_sc=False),
        out_type=jax.ShapeDtypeStruct((_NW, n_chunks, _CHUNK), jnp.float32),
        scratch_types=dict(
            src_idx=pltpu.VMEM((n_chunks, _CHUNK), jnp.int32),
            dst_idx=pltpu.VMEM((n_chunks, _CHUNK), jnp.int32),
            rs_bufs=[pltpu.VMEM((_CHUNK, 64), jnp.int32) for _ in range(_RING)],
            rd_bufs=[pltpu.VMEM((_CHUNK, 64), jnp.int32) for _ in range(_RING)],
            out_bufs=[pltpu.VMEM((_CHUNK,), jnp.float32) for _ in range(_RING)],
            sems=[pltpu.SemaphoreType.DMA for _ in range(_RING)],
            osems=[pltpu.SemaphoreType.DMA for _ in range(_RING)],
        ),
    )
    def body(z_hbm, src_hbm, dst_hbm, out_hbm,
             src_idx, dst_idx, rs_bufs, rd_bufs, out_bufs, sems, osems):
        wid = lax.axis_index("s") * _NC + lax.axis_index("c")

        pltpu.sync_copy(src_hbm.at[wid], src_idx)
        pltpu.sync_copy(dst_hbm.at[wid], dst_idx)

        def issue(l, b):
            pltpu.async_copy(z_hbm.at[src_idx.at[l]], rs_bufs[b], sems[b])
            pltpu.async_copy(z_hbm.at[dst_idx.at[l]], rd_bufs[b], sems[b])

        def drain(l, b):
            pltpu.make_async_copy(z_hbm.at[src_idx.at[l]], rs_bufs[b], sems[b]).wait()
            pltpu.make_async_copy(z_hbm.at[dst_idx.at[l]], rd_bufs[b], sems[b]).wait()

        def issue_out(l, b):
            pltpu.async_copy(out_bufs[b], out_hbm.at[wid, l], osems[b])

        def drain_out(b):
            pltpu.make_async_copy(out_bufs[b], out_hbm.at[wid, 0],
                                  osems[b]).wait()

        def compute(l, rs, rd, b):
            out_row = out_bufs[b]
            zero = jnp.zeros((_LANES,), jnp.float32)
            iota = lax.iota(jnp.int32, _LANES)

            def group(g, gcarry):
                row = jnp.full((_LANES,), g * _LANES, jnp.int32) + iota

                def dstep(i, accs):
                    a0, a1, a2, a3 = accs
                    cb = jnp.full((_LANES,), i * 16, jnp.int32) + iota
                    prods = []
                    for k in range(16):
                        col = (cb + k) & 63
                        s = plsc.load_gather(rs, [row, col])
                        t = plsc.load_gather(rd, [row, col])
                        p = (plsc.bitcast(s, jnp.bfloat16)
                             * plsc.bitcast(t, jnp.bfloat16))
                        p0, p1 = plsc.unpack(p, format=plsc.PackFormat.INTERLEAVED)
                        prods.append(p0 + p1)
                    for k in range(0, 16, 4):
                        a0 = a0 + prods[k]
                        a1 = a1 + prods[k + 1]
                        a2 = a2 + prods[k + 2]
                        a3 = a3 + prods[k + 3]
                    return a0, a1, a2, a3

                a0, a1, a2, a3 = lax.fori_loop(0, 64 // 16, dstep,
                                               (zero, zero, zero, zero))
                acc = (a0 + a1) + (a2 + a3)
                out_row[pl.ds(g * _LANES, _LANES)] = 1.0 / (1.0 + jnp.exp(-acc))
                return gcarry

            lax.fori_loop(0, _CHUNK // _LANES, group, 0)

        for b in range(_RING):
            issue(b, b)

        def quad(j, carry):
            for b in range(_RING):
                l = _RING * j + b
                drain(l, b)

                @pl.when(j >= 1)
                def _():
                    drain_out(b)

                compute(l, rs_bufs[b], rd_bufs[b], b)
                issue_out(l, b)

                @pl.when(l + _RING < n_chunks)
                def _():
                    issue(l + _RING, b)

            return carry

        lax.fori_loop(0, n_chunks // _RING, quad, 0)
        last = n_chunks - 1
        lb = last % _RING
        drain(last, lb)
        drain_out(lb)
        compute(last, rs_bufs[lb], rd_bufs[lb], lb)
        issue_out(last, lb)
        for b in range(_RING):
            drain_out(b)

    return body(z, src2d, dst2d)


def kernel(z, edge_index):
    n_edges = edge_index.shape[1]
    n_chunks = n_edges // (_NW * _CHUNK)
    ei = edge_index.astype(jnp.int32)
    src3d = ei[0].reshape(_NW, n_chunks, _CHUNK)
    dst3d = ei[1].reshape(_NW, n_chunks, _CHUNK)
    n_nodes, d_model = z.shape
    z_pk = jax.lax.bitcast_convert_type(
        z.astype(jnp.bfloat16).reshape(n_nodes, d_model // 2, 2), jnp.int32)
    out3d = _decode(z_pk, src3d, dst3d, n_edges=n_edges, n_chunks=n_chunks)
    return out3d.reshape(n_edges)


# speedup vs baseline: 1.2404x; 1.0027x over previous
import functools

import jax
import jax.numpy as jnp
from jax import lax
from jax.experimental import pallas as pl
from jax.experimental.pallas import tpu as pltpu
from jax.experimental.pallas import tpu_sc as plsc

_LANES = 16
_NW = 32
_NC = 2
_CHUNK = 80
_RING = 2


@functools.partial(jax.jit, static_argnames=("n_edges", "n_chunks"))
def _decode(z, src2d, dst2d, *, n_edges, n_chunks):
    mesh = plsc.VectorSubcoreMesh(core_axis_name="c", subcore_axis_name="s")

    @functools.partial(
        pl.kernel,
        mesh=mesh,
        compiler_params=pltpu.CompilerParams(needs_layout_passes=False,
                                             use_tc_tiling_on_sc=False),
        out_type=jax.ShapeDtypeStruct((_NW, n_chunks, _CHUNK), jnp.float32),
        scratch_types=dict(
            src_idx=pltpu.VMEM((n_chunks, _CHUNK), jnp.int32),
            dst_idx=pltpu.VMEM((n_chunks, _CHUNK), jnp.int32),
            rs_bufs=[pltpu.VMEM((_CHUNK, 64), jnp.int32) for _ in range(_RING)],
            rd_bufs=[pltpu.VMEM((_CHUNK, 64), jnp.int32) for _ in range(_RING)],
            z_sh=pltpu.VMEM_SHARED((10000, 64), jnp.int32),
            out_bufs=[pltpu.VMEM((_CHUNK,), jnp.float32) for _ in range(_RING)],
            sems=[pltpu.SemaphoreType.DMA for _ in range(_RING)],
            osems=[pltpu.SemaphoreType.DMA for _ in range(_RING)],
        ),
    )
    def body(z_hbm, src_hbm, dst_hbm, out_hbm,
             src_idx, dst_idx, z_sh, rs_bufs, rd_bufs, out_bufs, sems, osems):
        sid = lax.axis_index("s")
        wid = sid * _NC + lax.axis_index("c")

        pltpu.sync_copy(z_hbm.at[pl.ds(sid * 625, 625)],
                        z_sh.at[pl.ds(sid * 625, 625)])
        pltpu.sync_copy(src_hbm.at[wid], src_idx)
        pltpu.sync_copy(dst_hbm.at[wid], dst_idx)
        plsc.subcore_barrier()

        def issue(l, b):
            pltpu.async_copy(z_sh.at[src_idx.at[l]], rs_bufs[b], sems[b])
            pltpu.async_copy(z_sh.at[dst_idx.at[l]], rd_bufs[b], sems[b])

        def drain(l, b):
            pltpu.make_async_copy(z_sh.at[src_idx.at[l]], rs_bufs[b], sems[b]).wait()
            pltpu.make_async_copy(z_sh.at[dst_idx.at[l]], rd_bufs[b], sems[b]).wait()

        def issue_out(l, b):
            pltpu.async_copy(out_bufs[b], out_hbm.at[wid, l], osems[b])

        def drain_out(b):
            pltpu.make_async_copy(out_bufs[b], out_hbm.at[wid, 0],
                                  osems[b]).wait()

        def compute(l, rs, rd, b):
            out_row = out_bufs[b]
            zero = jnp.zeros((_LANES,), jnp.float32)
            iota = lax.iota(jnp.int32, _LANES)

            def group(g, gcarry):
                row = jnp.full((_LANES,), g * _LANES, jnp.int32) + iota

                def dstep(i, accs):
                    a0, a1, a2, a3 = accs
                    cb = jnp.full((_LANES,), i * 16, jnp.int32) + iota
                    prods = []
                    for k in range(16):
                        col = (cb + k) & 63
                        s = plsc.load_gather(rs, [row, col])
                        t = plsc.load_gather(rd, [row, col])
                        p = (plsc.bitcast(s, jnp.bfloat16)
                             * plsc.bitcast(t, jnp.bfloat16))
                        p0, p1 = plsc.unpack(p, format=plsc.PackFormat.INTERLEAVED)
                        prods.append(p0 + p1)
                    for k in range(0, 16, 4):
                        a0 = a0 + prods[k]
                        a1 = a1 + prods[k + 1]
                        a2 = a2 + prods[k + 2]
                        a3 = a3 + prods[k + 3]
                    return a0, a1, a2, a3

                a0, a1, a2, a3 = lax.fori_loop(0, 64 // 16, dstep,
                                               (zero, zero, zero, zero))
                acc = (a0 + a1) + (a2 + a3)
                out_row[pl.ds(g * _LANES, _LANES)] = 1.0 / (1.0 + jnp.exp(-acc))
                return gcarry

            lax.fori_loop(0, _CHUNK // _LANES, group, 0)

        for b in range(_RING):
            issue(b, b)

        def quad(j, carry):
            for b in range(_RING):
                l = _RING * j + b
                drain(l, b)

                @pl.when(j >= 1)
                def _():
                    drain_out(b)

                compute(l, rs_bufs[b], rd_bufs[b], b)
                issue_out(l, b)

                @pl.when(l + _RING < n_chunks)
                def _():
                    issue(l + _RING, b)

            return carry

        lax.fori_loop(0, n_chunks // _RING, quad, 0)
        last = n_chunks - 1
        lb = last % _RING
        drain(last, lb)
        drain_out(lb)
        compute(last, rs_bufs[lb], rd_bufs[lb], lb)
        issue_out(last, lb)
        for b in range(_RING):
            drain_out(b)

    return body(z, src2d, dst2d)


def kernel(z, edge_index):
    n_edges = edge_index.shape[1]
    n_chunks = n_edges // (_NW * _CHUNK)
    ei = edge_index.astype(jnp.int32)
    src3d = ei[0].reshape(_NW, n_chunks, _CHUNK)
    dst3d = ei[1].reshape(_NW, n_chunks, _CHUNK)
    n_nodes, d_model = z.shape
    z_pk = jax.lax.bitcast_convert_type(
        z.astype(jnp.bfloat16).reshape(n_nodes, d_model // 2, 2), jnp.int32)
    out3d = _decode(z_pk, src3d, dst3d, n_edges=n_edges, n_chunks=n_chunks)
    return out3d.reshape(n_edges)
